# Initial kernel scaffold; baseline (speedup 1.0000x reference)
import functools

import jax
import jax.numpy as jnp
from jax import lax
from jax.experimental import pallas as pl
from jax.experimental.pallas import tpu as pltpu
from jax.experimental.pallas import tpu_sc as plsc

N = 10000
E = 160000
F = 256
FH = 128
HOPS = 6
NC, NS, L = 2, 16, 16
CH = 128
NCH = (E + NS * CH - 1) // (NS * CH)
EPS = NCH * CH
EPAD = EPS * NS
NPAD = N + L
ZRO = NPAD // NS
WRO = N // NS
RB = 1000

_mesh = plsc.VectorSubcoreMesh(
    core_axis_name="c", subcore_axis_name="s", num_cores=NC, num_subcores=NS)


def _fill(buf, rows, val):
    v = jnp.full((L,), val, jnp.float32)

    @pl.loop(0, rows)
    def _(r):
        for c in range(buf.shape[1] // L):
            buf[r, pl.ds(c * L, L)] = v


@functools.partial(
    pl.kernel,
    out_type=jax.ShapeDtypeStruct((NPAD, L), jnp.float32),
    mesh=_mesh,
    scratch_types=[
        pltpu.VMEM_SHARED((NPAD, L), jnp.float32),
        pltpu.VMEM((NCH, CH), jnp.int32),
        pltpu.VMEM((CH, L), jnp.float32),
    ],
)
def _deg_kernel(dst3_hbm, out_hbm, hist, didx, ones_v):
    c = lax.axis_index("c")
    s = lax.axis_index("s")

    @pl.when(c == 0)
    def _():
        _fill(ones_v, CH, 0.0)
        for t in range(4):
            pltpu.sync_copy(ones_v, hist.at[pl.ds(s * ZRO + t * CH, CH)])
        pltpu.sync_copy(ones_v.at[pl.ds(0, ZRO - 4 * CH)],
                        hist.at[pl.ds(s * ZRO + 4 * CH, ZRO - 4 * CH)])
        _fill(ones_v, CH, 1.0)
        pltpu.sync_copy(dst3_hbm.at[s], didx)
        plsc.subcore_barrier()

        @pl.loop(0, NCH)
        def _(j):
            pltpu.sync_copy(ones_v, hist.at[didx.at[j]], add=True)

        plsc.subcore_barrier()
        pltpu.sync_copy(hist.at[pl.ds(s * ZRO, ZRO)],
                        out_hbm.at[pl.ds(s * ZRO, ZRO)])


@functools.partial(
    pl.kernel,
    out_type=jax.ShapeDtypeStruct((NC, N, FH), jnp.float32),
    mesh=_mesh,
    scratch_types=[
        pltpu.VMEM_SHARED((NPAD, FH), jnp.float32),
        pltpu.VMEM((NCH, CH), jnp.int32),
        pltpu.VMEM((NCH, CH), jnp.int32),
        pltpu.VMEM((CH, FH), jnp.float32),
        pltpu.SemaphoreType.DMA,
    ],
)
def _hop_kernel(g_hbm, src3_hbm, dst3_hbm, r_hbm, acc, sidx, didx, rows, sem):
    c = lax.axis_index("c")
    s = lax.axis_index("s")

    _fill(rows, CH, 0.0)
    for t in range(4):
        pltpu.sync_copy(rows, acc.at[pl.ds(s * ZRO + t * CH, CH)])
    pltpu.sync_copy(rows.at[pl.ds(0, ZRO - 4 * CH)],
                    acc.at[pl.ds(s * ZRO + 4 * CH, ZRO - 4 * CH)])
    pltpu.sync_copy(src3_hbm.at[s], sidx)
    pltpu.sync_copy(dst3_hbm.at[s], didx)
    plsc.subcore_barrier()

    def body(ghalf, rhalf):
        @pl.loop(0, NCH)
        def _(j):
            pltpu.async_copy(ghalf.at[sidx.at[j]], rows, sem).wait()
            pltpu.sync_copy(rows, acc.at[didx.at[j]], add=True)

        plsc.subcore_barrier()
        pltpu.sync_copy(acc.at[pl.ds(s * WRO, WRO)],
                        rhalf.at[pl.ds(s * WRO, WRO)])

    @pl.when(c == 0)
    def _():
        body(g_hbm.at[0], r_hbm.at[0])

    @pl.when(c == 1)
    def _():
        body(g_hbm.at[1], r_hbm.at[1])


def _mm1(x, w0, b0):
    def body(x_ref, w_ref, b_ref, o_ref):
        o_ref[...] = jnp.maximum(
            jnp.dot(x_ref[...], w_ref[...],
                    preferred_element_type=jnp.float32) + b_ref[...], 0.0)

    return pl.pallas_call(
        body,
        grid=(N // RB,),
        in_specs=[
            pl.BlockSpec((RB, F), lambda i: (i, 0)),
            pl.BlockSpec((F, F), lambda i: (0, 0)),
            pl.BlockSpec((1, F), lambda i: (0, 0)),
        ],
        out_specs=pl.BlockSpec((RB, F), lambda i: (i, 0)),
        out_shape=jax.ShapeDtypeStruct((N, F), jnp.float32),
    )(x, w0, b0.reshape(1, F))


def _prep(cnt, h0):
    def body(c_ref, h_ref, g_ref, i2_ref, sq_ref):
        deg = c_ref[...] + 1.0
        isd = lax.rsqrt(deg)
        i2_ref[...] = 1.0 / deg
        sq_ref[...] = jnp.sqrt(deg)
        h = h_ref[...]
        g_ref[0] = isd * h[:, :FH]
        g_ref[1] = isd * h[:, FH:]

    return pl.pallas_call(
        body,
        grid=(N // RB,),
        in_specs=[
            pl.BlockSpec((RB, 1), lambda i: (i, 0)),
            pl.BlockSpec((RB, F), lambda i: (i, 0)),
        ],
        out_specs=[
            pl.BlockSpec((NC, RB, FH), lambda i: (0, i, 0)),
            pl.BlockSpec((RB, 1), lambda i: (i, 0)),
            pl.BlockSpec((RB, 1), lambda i: (i, 0)),
        ],
        out_shape=[
            jax.ShapeDtypeStruct((NC, N, FH), jnp.float32),
            jax.ShapeDtypeStruct((N, 1), jnp.float32),
            jax.ShapeDtypeStruct((N, 1), jnp.float32),
        ],
    )(cnt, h0)


def _scale(isd2, r, g):
    def body(i2_ref, r_ref, g_ref, o_ref):
        o_ref[...] = i2_ref[...][None] * (r_ref[...] + g_ref[...])

    return pl.pallas_call(
        body,
        grid=(N // RB,),
        in_specs=[
            pl.BlockSpec((RB, 1), lambda i: (i, 0)),
            pl.BlockSpec((NC, RB, FH), lambda i: (0, i, 0)),
            pl.BlockSpec((NC, RB, FH), lambda i: (0, i, 0)),
        ],
        out_specs=pl.BlockSpec((NC, RB, FH), lambda i: (0, i, 0)),
        out_shape=jax.ShapeDtypeStruct((NC, N, FH), jnp.float32),
    )(isd2, r, g)


def _final(sqd, w_r, b, gs):
    def body(sq_ref, w_ref, b_ref, *refs):
        g_refs, o_ref = refs[:-1], refs[-1]
        sq = sq_ref[...]
        acc = jnp.zeros((RB, F), jnp.float32) + b_ref[...]
        for k in range(1 + HOPS):
            gk = g_refs[k][...]
            for h in range(NC):
                acc += jnp.dot(sq * gk[h], w_ref[k, h],
                               preferred_element_type=jnp.float32)
        o_ref[...] = jnp.maximum(acc, 0.0)

    g_spec = pl.BlockSpec((NC, RB, FH), lambda i: (0, i, 0))
    return pl.pallas_call(
        body,
        grid=(N // RB,),
        in_specs=[
            pl.BlockSpec((RB, 1), lambda i: (i, 0)),
            pl.BlockSpec((1 + HOPS, NC, FH, F), lambda i: (0, 0, 0, 0)),
            pl.BlockSpec((1, F), lambda i: (0, 0)),
        ] + [g_spec] * (1 + HOPS),
        out_specs=pl.BlockSpec((RB, F), lambda i: (i, 0)),
        out_shape=jax.ShapeDtypeStruct((N, F), jnp.float32),
    )(sqd, w_r, b, *gs)


def kernel(edge_index, features, W0, b0, W_rn, b_rn):
    src = edge_index[0].astype(jnp.int32)
    dst = edge_index[1].astype(jnp.int32)
    pad = EPAD - E
    src3 = jnp.concatenate(
        [src, jnp.zeros((pad,), jnp.int32)]).reshape(NS, NCH, CH)
    dst3 = jnp.concatenate(
        [dst, N + jnp.arange(pad, dtype=jnp.int32) % L]).reshape(NS, NCH, CH)

    degh = _deg_kernel(dst3)
    h0 = _mm1(features, W0, b0)
    g, isd2, sqd = _prep(degh[:N, :1], h0)

    gs = [g]
    for _ in range(HOPS):
        r = _hop_kernel(g, src3, dst3)
        g = _scale(isd2, r, g)
        gs.append(g)

    w_r = W_rn.reshape(1 + HOPS, NC, FH, F)
    return _final(sqd, w_r, b_rn.reshape(1, F), gs)


# speedup vs baseline: 6.9590x; 6.9590x over previous
import functools

import jax
import jax.numpy as jnp
from jax import lax
from jax.experimental import pallas as pl
from jax.experimental.pallas import tpu as pltpu
from jax.experimental.pallas import tpu_sc as plsc

N = 10000
E = 160000
F = 256
FH = 128
HOPS = 6
NC, NS, L = 2, 16, 16
CH = 128
NCH = (E + NS * CH - 1) // (NS * CH)
EPS = NCH * CH
EPAD = EPS * NS
NPAD = 10112
ZRO = NPAD // NS
WRO = 632
WLAST = N - (NS - 1) * WRO
RB = 1000

_mesh = plsc.VectorSubcoreMesh(
    core_axis_name="c", subcore_axis_name="s", num_cores=NC, num_subcores=NS)


def _fill(buf, rows, val):
    v = jnp.full((L,), val, jnp.float32)

    @pl.loop(0, rows)
    def _(r):
        for c in range(buf.shape[1] // L):
            buf[r, pl.ds(c * L, L)] = v


@functools.partial(
    pl.kernel,
    out_type=jax.ShapeDtypeStruct((NPAD, L), jnp.float32),
    mesh=_mesh,
    scratch_types=[
        pltpu.VMEM_SHARED((NPAD, L), jnp.float32),
        pltpu.VMEM((NCH, CH), jnp.int32),
        pltpu.VMEM((CH, L), jnp.float32),
    ],
)
def _deg_kernel(dst3_hbm, out_hbm, hist, didx, ones_v):
    c = lax.axis_index("c")
    s = lax.axis_index("s")

    @pl.when(c == 0)
    def _():
        _fill(ones_v, CH, 0.0)
        for t in range(4):
            pltpu.sync_copy(ones_v, hist.at[pl.ds(s * ZRO + t * CH, CH)])
        pltpu.sync_copy(ones_v.at[pl.ds(0, ZRO - 4 * CH)],
                        hist.at[pl.ds(s * ZRO + 4 * CH, ZRO - 4 * CH)])
        _fill(ones_v, CH, 1.0)
        pltpu.sync_copy(dst3_hbm.at[s], didx)
        plsc.subcore_barrier()

        @pl.loop(0, NCH)
        def _(j):
            pltpu.sync_copy(ones_v, hist.at[didx.at[j]], add=True)

        plsc.subcore_barrier()
        pltpu.sync_copy(hist.at[pl.ds(s * ZRO, ZRO)],
                        out_hbm.at[pl.ds(s * ZRO, ZRO)])


@functools.partial(
    pl.kernel,
    out_type=jax.ShapeDtypeStruct((NC, N, FH), jnp.float32),
    mesh=_mesh,
    scratch_types=[
        pltpu.VMEM_SHARED((NPAD, FH), jnp.float32),
        pltpu.VMEM((NCH, CH), jnp.int32),
        pltpu.VMEM((NCH, CH), jnp.int32),
        pltpu.VMEM((CH, FH), jnp.float32),
        pltpu.SemaphoreType.DMA,
    ],
)
def _hop_kernel(g_hbm, src3_hbm, dst3_hbm, r_hbm, acc, sidx, didx, rows, sem):
    c = lax.axis_index("c")
    s = lax.axis_index("s")

    _fill(rows, CH, 0.0)
    for t in range(4):
        pltpu.sync_copy(rows, acc.at[pl.ds(s * ZRO + t * CH, CH)])
    pltpu.sync_copy(rows.at[pl.ds(0, ZRO - 4 * CH)],
                    acc.at[pl.ds(s * ZRO + 4 * CH, ZRO - 4 * CH)])
    pltpu.sync_copy(src3_hbm.at[s], sidx)
    pltpu.sync_copy(dst3_hbm.at[s], didx)
    plsc.subcore_barrier()

    def body(ghalf, rhalf):
        @pl.loop(0, NCH)
        def _(j):
            pltpu.async_copy(ghalf.at[sidx.at[j]], rows, sem).wait()
            pltpu.sync_copy(rows, acc.at[didx.at[j]], add=True)

        plsc.subcore_barrier()

        @pl.when(s < NS - 1)
        def _():
            pltpu.sync_copy(acc.at[pl.ds(s * WRO, WRO)],
                            rhalf.at[pl.ds(s * WRO, WRO)])

        @pl.when(s == NS - 1)
        def _():
            pltpu.sync_copy(acc.at[pl.ds((NS - 1) * WRO, WLAST)],
                            rhalf.at[pl.ds((NS - 1) * WRO, WLAST)])

    @pl.when(c == 0)
    def _():
        body(g_hbm.at[0], r_hbm.at[0])

    @pl.when(c == 1)
    def _():
        body(g_hbm.at[1], r_hbm.at[1])


def _mm1(x, w0, b0):
    def body(x_ref, w_ref, b_ref, o_ref):
        o_ref[...] = jnp.maximum(
            jnp.dot(x_ref[...], w_ref[...],
                    preferred_element_type=jnp.float32) + b_ref[...], 0.0)

    return pl.pallas_call(
        body,
        grid=(N // RB,),
        in_specs=[
            pl.BlockSpec((RB, F), lambda i: (i, 0)),
            pl.BlockSpec((F, F), lambda i: (0, 0)),
            pl.BlockSpec((1, F), lambda i: (0, 0)),
        ],
        out_specs=pl.BlockSpec((RB, F), lambda i: (i, 0)),
        out_shape=jax.ShapeDtypeStruct((N, F), jnp.float32),
    )(x, w0, b0.reshape(1, F))


def _prep(cnt, h0):
    def body(c_ref, h_ref, g_ref, i2_ref, sq_ref):
        deg = c_ref[...] + 1.0
        isd = lax.rsqrt(deg)
        i2_ref[...] = 1.0 / deg
        sq_ref[...] = jnp.sqrt(deg)
        h = h_ref[...]
        g_ref[0] = isd * h[:, :FH]
        g_ref[1] = isd * h[:, FH:]

    return pl.pallas_call(
        body,
        grid=(N // RB,),
        in_specs=[
            pl.BlockSpec((RB, 1), lambda i: (i, 0)),
            pl.BlockSpec((RB, F), lambda i: (i, 0)),
        ],
        out_specs=[
            pl.BlockSpec((NC, RB, FH), lambda i: (0, i, 0)),
            pl.BlockSpec((RB, 1), lambda i: (i, 0)),
            pl.BlockSpec((RB, 1), lambda i: (i, 0)),
        ],
        out_shape=[
            jax.ShapeDtypeStruct((NC, N, FH), jnp.float32),
            jax.ShapeDtypeStruct((N, 1), jnp.float32),
            jax.ShapeDtypeStruct((N, 1), jnp.float32),
        ],
    )(cnt, h0)


def _scale(isd2, r, g):
    def body(i2_ref, r_ref, g_ref, o_ref):
        o_ref[...] = i2_ref[...][None] * (r_ref[...] + g_ref[...])

    return pl.pallas_call(
        body,
        grid=(N // RB,),
        in_specs=[
            pl.BlockSpec((RB, 1), lambda i: (i, 0)),
            pl.BlockSpec((NC, RB, FH), lambda i: (0, i, 0)),
            pl.BlockSpec((NC, RB, FH), lambda i: (0, i, 0)),
        ],
        out_specs=pl.BlockSpec((NC, RB, FH), lambda i: (0, i, 0)),
        out_shape=jax.ShapeDtypeStruct((NC, N, FH), jnp.float32),
    )(isd2, r, g)


def _final(sqd, w_r, b, gs):
    def body(sq_ref, w_ref, b_ref, *refs):
        g_refs, o_ref = refs[:-1], refs[-1]
        sq = sq_ref[...]
        acc = jnp.zeros((RB, F), jnp.float32) + b_ref[...]
        for k in range(1 + HOPS):
            gk = g_refs[k][...]
            for h in range(NC):
                acc += jnp.dot(sq * gk[h], w_ref[k, h],
                               preferred_element_type=jnp.float32)
        o_ref[...] = jnp.maximum(acc, 0.0)

    g_spec = pl.BlockSpec((NC, RB, FH), lambda i: (0, i, 0))
    return pl.pallas_call(
        body,
        grid=(N // RB,),
        in_specs=[
            pl.BlockSpec((RB, 1), lambda i: (i, 0)),
            pl.BlockSpec((1 + HOPS, NC, FH, F), lambda i: (0, 0, 0, 0)),
            pl.BlockSpec((1, F), lambda i: (0, 0)),
        ] + [g_spec] * (1 + HOPS),
        out_specs=pl.BlockSpec((RB, F), lambda i: (i, 0)),
        out_shape=jax.ShapeDtypeStruct((N, F), jnp.float32),
    )(sqd, w_r, b, *gs)


def kernel(edge_index, features, W0, b0, W_rn, b_rn):
    src = edge_index[0].astype(jnp.int32)
    dst = edge_index[1].astype(jnp.int32)
    pad = EPAD - E
    src3 = jnp.concatenate(
        [src, jnp.zeros((pad,), jnp.int32)]).reshape(NS, NCH, CH)
    dst3 = jnp.concatenate(
        [dst, N + jnp.arange(pad, dtype=jnp.int32) % L]).reshape(NS, NCH, CH)

    degh = _deg_kernel(dst3)
    h0 = _mm1(features, W0, b0)
    g, isd2, sqd = _prep(degh[:N, :1], h0)

    gs = [g]
    for _ in range(HOPS):
        r = _hop_kernel(g, src3, dst3)
        g = _scale(isd2, r, g)
        gs.append(g)

    w_r = W_rn.reshape(1 + HOPS, NC, FH, F)
    return _final(sqd, w_r, b_rn.reshape(1, F), gs)


# speedup vs baseline: 14.4030x; 2.0697x over previous
import functools

import jax
import jax.numpy as jnp
from jax import lax
from jax.experimental import pallas as pl
from jax.experimental.pallas import tpu as pltpu
from jax.experimental.pallas import tpu_sc as plsc

N = 10000
E = 160000
F = 256
FH = 128
HOPS = 6
NC, NS, L = 2, 16, 16
CH = 128
NBUF = 2
NCH = 80
GI = 16
NG = NCH // GI
EPS = NCH * CH
EPAD = EPS * NS
NPAD = 10112
ZRO = NPAD // NS
WRO = 632
WLAST = N - (NS - 1) * WRO
RB = 1000

_mesh = plsc.VectorSubcoreMesh(
    core_axis_name="c", subcore_axis_name="s", num_cores=NC, num_subcores=NS)


def _fill(buf, rows, val):
    v = jnp.full((L,), val, jnp.float32)

    @pl.loop(0, rows)
    def _(r):
        for c in range(buf.shape[1] // L):
            buf[r, pl.ds(c * L, L)] = v


@functools.partial(
    pl.kernel,
    out_type=jax.ShapeDtypeStruct((NPAD, L), jnp.float32),
    mesh=_mesh,
    scratch_types=[
        pltpu.VMEM_SHARED((NPAD, L), jnp.float32),
        pltpu.VMEM((NCH, CH), jnp.int32),
        pltpu.VMEM((CH, L), jnp.float32),
    ],
)
def _deg_kernel(dst3_hbm, out_hbm, hist, didx, ones_v):
    c = lax.axis_index("c")
    s = lax.axis_index("s")

    @pl.when(c == 0)
    def _():
        _fill(ones_v, CH, 0.0)
        for t in range(ZRO // CH):
            pltpu.sync_copy(ones_v, hist.at[pl.ds(s * ZRO + t * CH, CH)])
        pltpu.sync_copy(ones_v.at[pl.ds(0, ZRO % CH)],
                        hist.at[pl.ds(s * ZRO + ZRO - ZRO % CH, ZRO % CH)])
        _fill(ones_v, CH, 1.0)
        pltpu.sync_copy(dst3_hbm.at[s], didx)
        plsc.subcore_barrier()

        @pl.loop(0, NCH)
        def _(j):
            pltpu.sync_copy(ones_v, hist.at[didx.at[j]], add=True)

        plsc.subcore_barrier()
        pltpu.sync_copy(hist.at[pl.ds(s * ZRO, ZRO)],
                        out_hbm.at[pl.ds(s * ZRO, ZRO)])


@functools.partial(
    pl.kernel,
    out_type=jax.ShapeDtypeStruct((NC, N, FH), jnp.float32),
    mesh=_mesh,
    scratch_types=[
        pltpu.VMEM_SHARED((NPAD, FH), jnp.float32),
        pltpu.VMEM((NBUF, GI, CH), jnp.int32),
        pltpu.VMEM((NBUF, GI, CH), jnp.int32),
    ] + [pltpu.VMEM((CH, FH), jnp.float32)] * NBUF
      + [pltpu.SemaphoreType.DMA] * (3 * NBUF),
)
def _hop_kernel(g_hbm, src3_hbm, dst3_hbm, r_hbm, acc, sidx, didx, *bs):
    bufs = bs[:NBUF]
    semg = bs[NBUF:2 * NBUF]
    semsi = bs[2 * NBUF:2 * NBUF + NBUF]
    semdi = bs[2 * NBUF + NBUF:]
    c = lax.axis_index("c")
    s = lax.axis_index("s")

    _fill(bufs[0], CH, 0.0)
    for t in range(ZRO // CH):
        pltpu.sync_copy(bufs[0], acc.at[pl.ds(s * ZRO + t * CH, CH)])
    pltpu.sync_copy(bufs[0].at[pl.ds(0, ZRO % CH)],
                    acc.at[pl.ds(s * ZRO + ZRO - ZRO % CH, ZRO % CH)])

    def fetch_idx(g, p):
        pltpu.async_copy(src3_hbm.at[s].at[pl.ds(g * GI, GI)], sidx.at[p],
                         semsi[p])
        pltpu.async_copy(dst3_hbm.at[s].at[pl.ds(g * GI, GI)], didx.at[p],
                         semdi[p])

    def wait_idx_s(p):
        pltpu.make_async_copy(src3_hbm.at[s].at[pl.ds(0, GI)], sidx.at[p],
                              semsi[p]).wait()

    def wait_idx_d(p):
        pltpu.make_async_copy(dst3_hbm.at[s].at[pl.ds(0, GI)], didx.at[p],
                              semdi[p]).wait()

    fetch_idx(0, 0)
    wait_idx_s(0)
    wait_idx_d(0)
    plsc.subcore_barrier()

    def body(ghalf, rhalf):
        def wait_gather(b):
            pltpu.make_async_copy(ghalf.at[pl.ds(0, CH)], bufs[b],
                                  semg[b]).wait()

        def start_gather(p, b, buf):
            pltpu.async_copy(ghalf.at[sidx.at[p, b]], bufs[buf], semg[buf])

        def group(g, p, is_last):
            if not is_last:
                fetch_idx(g + 1, 1 - p)
            for b in range(GI):
                bb = b % NBUF
                wait_gather(bb)
                pltpu.sync_copy(bufs[bb], acc.at[didx.at[p, b]], add=True)
                nb = b + NBUF
                if nb < GI:
                    start_gather(p, nb, bb)
                elif not is_last:
                    if nb == GI:
                        wait_idx_s(1 - p)
                    start_gather(1 - p, nb - GI, bb)
            if not is_last:
                wait_idx_d(1 - p)

        for b in range(NBUF):
            start_gather(0, b, b)

        @pl.loop(0, (NG - 1) // 2)
        def _(t):
            group(2 * t, 0, False)
            group(2 * t + 1, 1, False)

        group(NG - 1, 0, True)

        plsc.subcore_barrier()

        @pl.when(s < NS - 1)
        def _():
            pltpu.sync_copy(acc.at[pl.ds(s * WRO, WRO)],
                            rhalf.at[pl.ds(s * WRO, WRO)])

        @pl.when(s == NS - 1)
        def _():
            pltpu.sync_copy(acc.at[pl.ds((NS - 1) * WRO, WLAST)],
                            rhalf.at[pl.ds((NS - 1) * WRO, WLAST)])

    @pl.when(c == 0)
    def _():
        body(g_hbm.at[0], r_hbm.at[0])

    @pl.when(c == 1)
    def _():
        body(g_hbm.at[1], r_hbm.at[1])


def _mm1(x, w0, b0):
    def body(x_ref, w_ref, b_ref, o_ref):
        o_ref[...] = jnp.maximum(
            jnp.dot(x_ref[...], w_ref[...],
                    preferred_element_type=jnp.float32) + b_ref[...], 0.0)

    return pl.pallas_call(
        body,
        grid=(N // RB,),
        in_specs=[
            pl.BlockSpec((RB, F), lambda i: (i, 0)),
            pl.BlockSpec((F, F), lambda i: (0, 0)),
            pl.BlockSpec((1, F), lambda i: (0, 0)),
        ],
        out_specs=pl.BlockSpec((RB, F), lambda i: (i, 0)),
        out_shape=jax.ShapeDtypeStruct((N, F), jnp.float32),
    )(x, w0, b0.reshape(1, F))


def _prep(cnt, h0):
    def body(c_ref, h_ref, g_ref, i2_ref, sq_ref):
        deg = c_ref[...] + 1.0
        isd = lax.rsqrt(deg)
        i2_ref[...] = 1.0 / deg
        sq_ref[...] = jnp.sqrt(deg)
        h = h_ref[...]
        g_ref[0] = isd * h[:, :FH]
        g_ref[1] = isd * h[:, FH:]

    return pl.pallas_call(
        body,
        grid=(N // RB,),
        in_specs=[
            pl.BlockSpec((RB, 1), lambda i: (i, 0)),
            pl.BlockSpec((RB, F), lambda i: (i, 0)),
        ],
        out_specs=[
            pl.BlockSpec((NC, RB, FH), lambda i: (0, i, 0)),
            pl.BlockSpec((RB, 1), lambda i: (i, 0)),
            pl.BlockSpec((RB, 1), lambda i: (i, 0)),
        ],
        out_shape=[
            jax.ShapeDtypeStruct((NC, N, FH), jnp.float32),
            jax.ShapeDtypeStruct((N, 1), jnp.float32),
            jax.ShapeDtypeStruct((N, 1), jnp.float32),
        ],
    )(cnt, h0)


def _scale(isd2, r, g):
    def body(i2_ref, r_ref, g_ref, o_ref):
        o_ref[...] = i2_ref[...][None] * (r_ref[...] + g_ref[...])

    return pl.pallas_call(
        body,
        grid=(N // RB,),
        in_specs=[
            pl.BlockSpec((RB, 1), lambda i: (i, 0)),
            pl.BlockSpec((NC, RB, FH), lambda i: (0, i, 0)),
            pl.BlockSpec((NC, RB, FH), lambda i: (0, i, 0)),
        ],
        out_specs=pl.BlockSpec((NC, RB, FH), lambda i: (0, i, 0)),
        out_shape=jax.ShapeDtypeStruct((NC, N, FH), jnp.float32),
    )(isd2, r, g)


def _final(sqd, w_r, b, gs):
    def body(sq_ref, w_ref, b_ref, *refs):
        g_refs, o_ref = refs[:-1], refs[-1]
        sq = sq_ref[...]
        acc = jnp.zeros((RB, F), jnp.float32) + b_ref[...]
        for k in range(1 + HOPS):
            gk = g_refs[k][...]
            for h in range(NC):
                acc += jnp.dot(sq * gk[h], w_ref[k, h],
                               preferred_element_type=jnp.float32)
        o_ref[...] = jnp.maximum(acc, 0.0)

    g_spec = pl.BlockSpec((NC, RB, FH), lambda i: (0, i, 0))
    return pl.pallas_call(
        body,
        grid=(N // RB,),
        in_specs=[
            pl.BlockSpec((RB, 1), lambda i: (i, 0)),
            pl.BlockSpec((1 + HOPS, NC, FH, F), lambda i: (0, 0, 0, 0)),
            pl.BlockSpec((1, F), lambda i: (0, 0)),
        ] + [g_spec] * (1 + HOPS),
        out_specs=pl.BlockSpec((RB, F), lambda i: (i, 0)),
        out_shape=jax.ShapeDtypeStruct((N, F), jnp.float32),
    )(sqd, w_r, b, *gs)


def kernel(edge_index, features, W0, b0, W_rn, b_rn):
    src = edge_index[0].astype(jnp.int32)
    dst = edge_index[1].astype(jnp.int32)
    pad = EPAD - E
    prng = jnp.arange(pad, dtype=jnp.int32)
    src3 = jnp.concatenate(
        [src, (prng * 97) % N]).reshape(NS, NCH, CH)
    dst3 = jnp.concatenate(
        [dst, N + prng % (NPAD - N)]).reshape(NS, NCH, CH)

    degh = _deg_kernel(dst3)
    h0 = _mm1(features, W0, b0)
    g, isd2, sqd = _prep(degh[:N, :1], h0)

    gs = [g]
    for _ in range(HOPS):
        r = _hop_kernel(g, src3, dst3)
        g = _scale(isd2, r, g)
        gs.append(g)

    w_r = W_rn.reshape(1 + HOPS, NC, FH, F)
    return _final(sqd, w_r, b_rn.reshape(1, F), gs)
